# baseline (speedup 1.0000x reference)
import functools

import jax
import jax.numpy as jnp
from jax import lax
from jax.experimental import pallas as pl
from jax.experimental.pallas import tpu as pltpu
from jax.experimental.pallas import tpu_sc as plsc

N = 10000
E = 160000
IN_DIM = 128
HID = 64
HEADS = 4
H1 = HEADS * HID
EDGE_DIM = 5
G = 64

_RB = 1000
_EB = 2000
_CH = 128
_NW = 32
E_PAD = 163840
_CPW = E_PAD // _NW // _CH

_EPW = E // _NW
_SFULL = _EPW // _CH
_STAIL = _EPW - _SFULL * _CH

_W1 = 128
_W2 = 128
N_PAD = 10240
_NPT = N_PAD // 16


def _prologue_body(x_ref, gin_ref, bin_ref, Wl_ref, bl_ref, Wr_ref, br_ref,
                   Wres_ref, bres_ref, x0_ref, xl_ref, xr_ref, res_ref):
    x0 = x_ref[:] * gin_ref[:] + bin_ref[:]
    x0_ref[:] = x0
    xl_ref[:] = jnp.dot(x0, Wl_ref[:], preferred_element_type=jnp.float32) + bl_ref[:]
    xr_ref[:] = jnp.dot(x0, Wr_ref[:], preferred_element_type=jnp.float32) + br_ref[:]
    res_ref[:] = jnp.dot(x0, Wres_ref[:], preferred_element_type=jnp.float32) + bres_ref[:]


def _dense_prologue(x, gin_eff, bin_, Wl1, bl1, Wr1, br1, Wres, bres):
    return pl.pallas_call(
        _prologue_body,
        grid=(N // _RB,),
        in_specs=[
            pl.BlockSpec((_RB, IN_DIM), lambda i: (i, 0)),
            pl.BlockSpec((1, IN_DIM), lambda i: (0, 0)),
            pl.BlockSpec((1, IN_DIM), lambda i: (0, 0)),
            pl.BlockSpec((IN_DIM, H1), lambda i: (0, 0)),
            pl.BlockSpec((1, H1), lambda i: (0, 0)),
            pl.BlockSpec((IN_DIM, H1), lambda i: (0, 0)),
            pl.BlockSpec((1, H1), lambda i: (0, 0)),
            pl.BlockSpec((IN_DIM, HID), lambda i: (0, 0)),
            pl.BlockSpec((1, HID), lambda i: (0, 0)),
        ],
        out_specs=[
            pl.BlockSpec((_RB, IN_DIM), lambda i: (i, 0)),
            pl.BlockSpec((_RB, H1), lambda i: (i, 0)),
            pl.BlockSpec((_RB, H1), lambda i: (i, 0)),
            pl.BlockSpec((_RB, HID), lambda i: (i, 0)),
        ],
        out_shape=[
            jax.ShapeDtypeStruct((N, IN_DIM), jnp.float32),
            jax.ShapeDtypeStruct((N, H1), jnp.float32),
            jax.ShapeDtypeStruct((N, H1), jnp.float32),
            jax.ShapeDtypeStruct((N, HID), jnp.float32),
        ],
    )(x, gin_eff[None, :], bin_[None, :], Wl1, bl1[None, :], Wr1, br1[None, :],
      Wres, bres[None, :])


def _make_gather(DA, DB):
    mesh = plsc.VectorSubcoreMesh(core_axis_name="c", subcore_axis_name="s")

    @functools.partial(
        pl.kernel, mesh=mesh,
        out_type=[
            jax.ShapeDtypeStruct((E_PAD, DA), jnp.float32),
            jax.ShapeDtypeStruct((E_PAD, DB), jnp.float32),
        ],
        scratch_types=[
            pltpu.VMEM((_CH,), jnp.int32),
            pltpu.VMEM((_CH,), jnp.int32),
            pltpu.VMEM((_CH, DA), jnp.float32),
            pltpu.VMEM((_CH, DA), jnp.float32),
            pltpu.SemaphoreType.DMA,
            pltpu.SemaphoreType.DMA,
        ],
    )
    def k(ta_hbm, ia_hbm, tb_hbm, ib_hbm, oa_hbm, ob_hbm,
          idx0_v, idx1_v, a0_v, a1_v, sem0, sem1):
        wid = lax.axis_index("s") * 2 + lax.axis_index("c")
        wbase = wid * (_CPW * _CH)

        def run_table(t_hbm, i_hbm, o_hbm, buf0, buf1):
            pltpu.sync_copy(i_hbm.at[pl.ds(wbase, _CH)], idx0_v)
            pltpu.async_copy(t_hbm.at[idx0_v], buf0, sem0)

            def body(kk, _):
                c0 = kk * 2
                base0 = wbase + c0 * _CH
                base1 = base0 + _CH
                pltpu.sync_copy(i_hbm.at[pl.ds(base1, _CH)], idx1_v)
                pltpu.async_copy(t_hbm.at[idx1_v], buf1, sem1)
                pltpu.make_async_copy(t_hbm.at[idx0_v], buf0, sem0).wait()
                pltpu.sync_copy(buf0, o_hbm.at[pl.ds(base0, _CH)])

                @pl.when(kk < _CPW // 2 - 1)
                def _():
                    pltpu.sync_copy(i_hbm.at[pl.ds(base1 + _CH, _CH)], idx0_v)
                    pltpu.async_copy(t_hbm.at[idx0_v], buf0, sem0)

                pltpu.make_async_copy(t_hbm.at[idx1_v], buf1, sem1).wait()
                pltpu.sync_copy(buf1, o_hbm.at[pl.ds(base1, _CH)])
                return ()

            lax.fori_loop(0, _CPW // 2, body, (), unroll=False)

        run_table(ta_hbm, ia_hbm, oa_hbm, a0_v, a1_v)
        run_table(tb_hbm, ib_hbm, ob_hbm, a0_v, a1_v)

    return k


_gather1 = _make_gather(H1, H1)
_gather2 = _make_gather(2 * HID, 2 * HID)


def _scatter1(xjwp0, xjwp1, xjwp2, dst, zeros_hbm):
    mesh = plsc.VectorSubcoreMesh(core_axis_name="c", subcore_axis_name="s")

    @functools.partial(
        pl.kernel, mesh=mesh,
        out_type=jax.ShapeDtypeStruct((2, 3, N_PAD, _W1), jnp.float32),
        scratch_types=[
            pltpu.VMEM((_CH,), jnp.int32),
            pltpu.VMEM((_CH, _W1), jnp.float32),
            pltpu.VMEM((_STAIL,), jnp.int32),
            pltpu.VMEM((_STAIL, _W1), jnp.float32),
            pltpu.VMEM_SHARED((N_PAD, _W1), jnp.float32),
            pltpu.SemaphoreType.DMA,
        ],
    )
    def k(p0_hbm, p1_hbm, p2_hbm, d_hbm, z_hbm, out_hbm, idx_v, buf_v,
          idxt_v, buft_v, acc_sh, sem):
        cid = lax.axis_index("c")
        sid = lax.axis_index("s")
        wid = sid * 2 + cid
        wbase = wid * _EPW

        for ph, src_hbm in ((0, p0_hbm), (1, p1_hbm), (2, p2_hbm)):
            pltpu.sync_copy(z_hbm.at[pl.ds(sid * _NPT, _NPT)],
                            acc_sh.at[pl.ds(sid * _NPT, _NPT)])
            plsc.subcore_barrier()

            def body(c, _):
                base = wbase + c * _CH
                pltpu.sync_copy(d_hbm.at[pl.ds(base, _CH)], idx_v)
                pltpu.sync_copy(src_hbm.at[pl.ds(base, _CH)], buf_v)
                pltpu.sync_copy(buf_v, acc_sh.at[idx_v], add=True)
                return ()

            lax.fori_loop(0, _SFULL, body, (), unroll=False)

            tbase = wbase + _SFULL * _CH
            pltpu.sync_copy(d_hbm.at[pl.ds(tbase, _STAIL)], idxt_v)
            pltpu.sync_copy(src_hbm.at[pl.ds(tbase, _STAIL)], buft_v)
            pltpu.sync_copy(buft_v, acc_sh.at[idxt_v], add=True)

            plsc.subcore_barrier()
            pltpu.sync_copy(acc_sh.at[pl.ds(sid * _NPT, _NPT)],
                            out_hbm.at[cid, ph, pl.ds(sid * _NPT, _NPT)])
            plsc.subcore_barrier()

    return k(xjwp0, xjwp1, xjwp2, dst, zeros_hbm)


def _scatter2(xjwp2, dst, zeros_hbm):
    mesh = plsc.VectorSubcoreMesh(core_axis_name="c", subcore_axis_name="s")

    @functools.partial(
        pl.kernel, mesh=mesh,
        out_type=jax.ShapeDtypeStruct((2, N_PAD, _W2), jnp.float32),
        scratch_types=[
            pltpu.VMEM((_CH,), jnp.int32),
            pltpu.VMEM((_CH, _W2), jnp.float32),
            pltpu.VMEM((_STAIL,), jnp.int32),
            pltpu.VMEM((_STAIL, _W2), jnp.float32),
            pltpu.VMEM_SHARED((N_PAD, _W2), jnp.float32),
            pltpu.SemaphoreType.DMA,
        ],
    )
    def k(p_hbm, d_hbm, z_hbm, out_hbm, idx_v, buf_v, idxt_v, buft_v,
          acc_sh, sem):
        cid = lax.axis_index("c")
        sid = lax.axis_index("s")
        wid = sid * 2 + cid
        wbase = wid * _EPW

        pltpu.sync_copy(z_hbm.at[pl.ds(sid * _NPT, _NPT)],
                        acc_sh.at[pl.ds(sid * _NPT, _NPT)])
        plsc.subcore_barrier()

        def body(c, _):
            base = wbase + c * _CH
            pltpu.sync_copy(d_hbm.at[pl.ds(base, _CH)], idx_v)
            pltpu.sync_copy(p_hbm.at[pl.ds(base, _CH)], buf_v)
            pltpu.sync_copy(buf_v, acc_sh.at[idx_v], add=True)
            return ()

        lax.fori_loop(0, _SFULL, body, (), unroll=False)

        tbase = wbase + _SFULL * _CH
        pltpu.sync_copy(d_hbm.at[pl.ds(tbase, _STAIL)], idxt_v)
        pltpu.sync_copy(p_hbm.at[pl.ds(tbase, _STAIL)], buft_v)
        pltpu.sync_copy(buft_v, acc_sh.at[idxt_v], add=True)

        plsc.subcore_barrier()
        pltpu.sync_copy(acc_sh.at[pl.ds(sid * _NPT, _NPT)],
                        out_hbm.at[cid, pl.ds(sid * _NPT, _NPT)])
        plsc.subcore_barrier()

    return k(xjwp2, dst, zeros_hbm)


def _edge1_body(xj_ref, xi_ref, ea_ref, We_ref, att_ref, p0_ref, p1_ref, p2_ref):
    e = jnp.dot(ea_ref[:, :EDGE_DIM], We_ref[:],
                preferred_element_type=jnp.float32)
    z = xj_ref[:] + xi_ref[:] + e
    z = jnp.where(z > 0, z, 0.2 * z)
    B = z.shape[0]
    alpha = (z.reshape(B, HEADS, HID) * att_ref[:][None]).sum(-1)
    ex = jnp.exp(alpha)
    xjw = (ex[:, :, None] * xj_ref[:].reshape(B, HEADS, HID)).reshape(B, H1)
    p0_ref[:] = xjw[:, :128]
    p1_ref[:] = xjw[:, 128:]
    p2_ref[:] = jnp.concatenate(
        [ex, jnp.zeros((B, _W1 - HEADS), jnp.float32)], axis=1)


def _edge1(xj, xi, ea8, We1, att1):
    return pl.pallas_call(
        _edge1_body,
        grid=(E // _EB,),
        in_specs=[
            pl.BlockSpec((_EB, H1), lambda i: (i, 0)),
            pl.BlockSpec((_EB, H1), lambda i: (i, 0)),
            pl.BlockSpec((_EB, 8), lambda i: (i, 0)),
            pl.BlockSpec((EDGE_DIM, H1), lambda i: (0, 0)),
            pl.BlockSpec((HEADS, HID), lambda i: (0, 0)),
        ],
        out_specs=[
            pl.BlockSpec((_EB, _W1), lambda i: (i, 0)),
            pl.BlockSpec((_EB, _W1), lambda i: (i, 0)),
            pl.BlockSpec((_EB, _W1), lambda i: (i, 0)),
        ],
        out_shape=[
            jax.ShapeDtypeStruct((E, _W1), jnp.float32),
            jax.ShapeDtypeStruct((E, _W1), jnp.float32),
            jax.ShapeDtypeStruct((E, _W1), jnp.float32),
        ],
    )(xj, xi, ea8, We1, att1)


def _mid_body(p00_ref, p10_ref, p01_ref, p11_ref, p02_ref, p12_ref,
              xl1_ref, xr1_ref, pe0_ref, pe1_ref,
              We1_ref, att1_ref, bias_ref, g_ref, b_ref, Wl_ref, bl_ref,
              Wr_ref, br_ref, xb2_ref):
    B = p00_ref.shape[2]
    c0 = p00_ref[0, 0] + p10_ref[0, 0]
    c1 = p01_ref[0, 0] + p11_ref[0, 0]
    raw = jnp.concatenate([c0, c1], axis=1)
    s_edges = (p02_ref[0, 0] + p12_ref[0, 0])[:, :HEADS]
    pe = pe0_ref[0] + pe1_ref[0]
    eal = pe[:, :EDGE_DIM] / jnp.maximum(pe[:, EDGE_DIM:EDGE_DIM + 1], 1.0)
    el = jnp.dot(eal, We1_ref[:], preferred_element_type=jnp.float32)
    zs = xl1_ref[:] + xr1_ref[:] + el
    zs = jnp.where(zs > 0, zs, 0.2 * zs)
    exs = jnp.exp((zs.reshape(B, HEADS, HID) * att1_ref[:][None]).sum(-1))
    tot = raw.reshape(B, HEADS, HID) + exs[:, :, None] * xl1_ref[:].reshape(B, HEADS, HID)
    s = s_edges + exs + 1e-16
    h = (tot / s[:, :, None]).reshape(B, H1) + bias_ref[:]
    mu = h.mean(-1, keepdims=True)
    var = ((h - mu) ** 2).mean(-1, keepdims=True)
    hn = (h - mu) / jnp.sqrt(var + 1e-5) * g_ref[:] + b_ref[:]
    he = jnp.where(hn > 0, hn, jnp.exp(hn) - 1.0)
    xl2 = jnp.dot(he, Wl_ref[:], preferred_element_type=jnp.float32) + bl_ref[:]
    xr2 = jnp.dot(he, Wr_ref[:], preferred_element_type=jnp.float32) + br_ref[:]
    xb2_ref[:] = jnp.concatenate([xl2, xr2], axis=1)


def _mid(P1, xl1, xr1, Pea, We1, att1, bias1, g1, b1, Wl2, bl2, Wr2, br2):
    return pl.pallas_call(
        _mid_body,
        grid=(N // _RB,),
        in_specs=[
            pl.BlockSpec((1, 1, _RB, _W1), lambda i: (0, 0, i, 0)),
            pl.BlockSpec((1, 1, _RB, _W1), lambda i: (1, 0, i, 0)),
            pl.BlockSpec((1, 1, _RB, _W1), lambda i: (0, 1, i, 0)),
            pl.BlockSpec((1, 1, _RB, _W1), lambda i: (1, 1, i, 0)),
            pl.BlockSpec((1, 1, _RB, _W1), lambda i: (0, 2, i, 0)),
            pl.BlockSpec((1, 1, _RB, _W1), lambda i: (1, 2, i, 0)),
            pl.BlockSpec((_RB, H1), lambda i: (i, 0)),
            pl.BlockSpec((_RB, H1), lambda i: (i, 0)),
            pl.BlockSpec((1, _RB, _W2), lambda i: (0, i, 0)),
            pl.BlockSpec((1, _RB, _W2), lambda i: (1, i, 0)),
            pl.BlockSpec((EDGE_DIM, H1), lambda i: (0, 0)),
            pl.BlockSpec((HEADS, HID), lambda i: (0, 0)),
            pl.BlockSpec((1, H1), lambda i: (0, 0)),
            pl.BlockSpec((1, H1), lambda i: (0, 0)),
            pl.BlockSpec((1, H1), lambda i: (0, 0)),
            pl.BlockSpec((H1, HID), lambda i: (0, 0)),
            pl.BlockSpec((1, HID), lambda i: (0, 0)),
            pl.BlockSpec((H1, HID), lambda i: (0, 0)),
            pl.BlockSpec((1, HID), lambda i: (0, 0)),
        ],
        out_specs=pl.BlockSpec((_RB, 2 * HID), lambda i: (i, 0)),
        out_shape=jax.ShapeDtypeStruct((N, 2 * HID), jnp.float32),
    )(P1, P1, P1, P1, P1, P1, xl1, xr1, Pea, Pea, We1, att1, bias1[None, :],
      g1[None, :], b1[None, :], Wl2, bl2[None, :], Wr2, br2[None, :])


def _mid_wrap(P1, *args):
    return _mid(P1, *args)


def _edge2_body(xjf_ref, xif_ref, ea_ref, We_ref, att_ref, xjw_ref):
    e = jnp.dot(ea_ref[:, :EDGE_DIM], We_ref[:],
                preferred_element_type=jnp.float32)
    xj = xjf_ref[:, :HID]
    xi = xif_ref[:, HID:]
    z = xj + xi + e
    z = jnp.where(z > 0, z, 0.2 * z)
    alpha = (z * att_ref[:]).sum(-1, keepdims=True)
    ex = jnp.exp(alpha)
    B = z.shape[0]
    xjw_ref[:] = jnp.concatenate(
        [ex * xj, ex, jnp.zeros((B, _W2 - HID - 1), jnp.float32)], axis=1)


def _edge2(xjf, xif, ea8, We2, att2):
    return pl.pallas_call(
        _edge2_body,
        grid=(E // _EB,),
        in_specs=[
            pl.BlockSpec((_EB, 2 * HID), lambda i: (i, 0)),
            pl.BlockSpec((_EB, 2 * HID), lambda i: (i, 0)),
            pl.BlockSpec((_EB, 8), lambda i: (i, 0)),
            pl.BlockSpec((EDGE_DIM, HID), lambda i: (0, 0)),
            pl.BlockSpec((1, HID), lambda i: (0, 0)),
        ],
        out_specs=pl.BlockSpec((_EB, _W2), lambda i: (i, 0)),
        out_shape=jax.ShapeDtypeStruct((E, _W2), jnp.float32),
    )(xjf, xif, ea8, We2, att2)


def _epi_body(q0_ref, q1_ref, xb2_ref, pe0_ref, pe1_ref, We2_ref, att2_ref,
              bias_ref, g_ref, b_ref, res_ref, out_ref):
    raw = q0_ref[0] + q1_ref[0]
    xl2 = xb2_ref[:, :HID]
    xr2 = xb2_ref[:, HID:]
    pe = pe0_ref[0] + pe1_ref[0]
    eal = pe[:, :EDGE_DIM] / jnp.maximum(pe[:, EDGE_DIM:EDGE_DIM + 1], 1.0)
    el = jnp.dot(eal, We2_ref[:], preferred_element_type=jnp.float32)
    zs = xl2 + xr2 + el
    zs = jnp.where(zs > 0, zs, 0.2 * zs)
    exs = jnp.exp((zs * att2_ref[:]).sum(-1, keepdims=True))
    tot = raw[:, :HID] + exs * xl2
    s = raw[:, HID:HID + 1] + exs + 1e-16
    h = tot / s + bias_ref[:]
    mu = h.mean(-1, keepdims=True)
    var = ((h - mu) ** 2).mean(-1, keepdims=True)
    hn = (h - mu) / jnp.sqrt(var + 1e-5) * g_ref[:] + b_ref[:]
    he = jnp.where(hn > 0, hn, jnp.exp(hn) - 1.0)
    out_ref[:] = he + res_ref[:]


def _epilogue(P2, xb2, Pea, We2, att2, bias2, g2, b2n, res):
    return pl.pallas_call(
        _epi_body,
        grid=(N // _RB,),
        in_specs=[
            pl.BlockSpec((1, _RB, _W2), lambda i: (0, i, 0)),
            pl.BlockSpec((1, _RB, _W2), lambda i: (1, i, 0)),
            pl.BlockSpec((_RB, 2 * HID), lambda i: (i, 0)),
            pl.BlockSpec((1, _RB, _W2), lambda i: (0, i, 0)),
            pl.BlockSpec((1, _RB, _W2), lambda i: (1, i, 0)),
            pl.BlockSpec((EDGE_DIM, HID), lambda i: (0, 0)),
            pl.BlockSpec((1, HID), lambda i: (0, 0)),
            pl.BlockSpec((1, HID), lambda i: (0, 0)),
            pl.BlockSpec((1, HID), lambda i: (0, 0)),
            pl.BlockSpec((1, HID), lambda i: (0, 0)),
            pl.BlockSpec((_RB, HID), lambda i: (i, 0)),
        ],
        out_specs=pl.BlockSpec((_RB, HID), lambda i: (i, 0)),
        out_shape=jax.ShapeDtypeStruct((N, HID), jnp.float32),
    )(P2, P2, xb2, Pea, Pea, We2, att2[0:1], bias2[None, :], g2[None, :],
      b2n[None, :], res)


def kernel(x, edge_index, edge_attr, batch, gin, bin_, Wl1, bl1, Wr1, br1, We1,
           att1, bias1, g1, b1, Wl2, bl2, Wr2, br2, We2, att2, bias2, g2, b2n,
           Wres, bres):
    src = edge_index[0]
    dst = edge_index[1]
    pad = jnp.zeros((E_PAD - E,), jnp.int32)
    srcp = jnp.concatenate([src, pad])
    dstp = jnp.concatenate([dst, pad])
    zeros_nw1 = jnp.zeros((N_PAD, _W1), jnp.float32)

    ea8 = jnp.concatenate([edge_attr, jnp.zeros((E, 3), jnp.float32)], axis=1)
    ea128 = jnp.concatenate(
        [edge_attr, jnp.ones((E, 1), jnp.float32),
         jnp.zeros((E, _W2 - EDGE_DIM - 1), jnp.float32)], axis=1)

    gin_eff = gin / jnp.sqrt(jnp.float32(1.0 + 1e-5))
    x0, xl1, xr1, res = _dense_prologue(x, gin_eff, bin_, Wl1, bl1, Wr1, br1,
                                        Wres, bres)

    Pea = _scatter2(ea128, dst, zeros_nw1)

    xj1, xi1 = _gather1(xl1, srcp, xr1, dstp)
    xjwp0, xjwp1, xjwp2 = _edge1(xj1, xi1, ea8, We1, att1)
    P1 = _scatter1(xjwp0, xjwp1, xjwp2, dst, zeros_nw1)
    xb2 = _mid_wrap(P1, xl1, xr1, Pea, We1, att1, bias1, g1, b1, Wl2, bl2,
                    Wr2, br2)

    xjf2, xif2 = _gather2(xb2, srcp, xb2, dstp)
    xjw2 = _edge2(xjf2, xif2, ea8, We2, att2)
    P2 = _scatter2(xjw2, dst, zeros_nw1)
    node_emb = _epilogue(P2, xb2, Pea, We2, att2, bias2, g2, b2n, res)

    cnt = jax.ops.segment_sum(jnp.ones((N,), jnp.float32), batch, num_segments=G)
    mean = jax.ops.segment_sum(node_emb, batch, num_segments=G) / jnp.maximum(cnt, 1.0)[:, None]
    mx = jax.ops.segment_max(node_emb, batch, num_segments=G)
    mx = jnp.where(jnp.isfinite(mx), mx, 0.0)
    graph_emb = jnp.concatenate([mean, mx], axis=-1)
    return node_emb, graph_emb


# speedup vs baseline: 1.0776x; 1.0776x over previous
import functools

import jax
import jax.numpy as jnp
from jax import lax
from jax.experimental import pallas as pl
from jax.experimental.pallas import tpu as pltpu
from jax.experimental.pallas import tpu_sc as plsc

N = 10000
E = 160000
IN_DIM = 128
HID = 64
HEADS = 4
H1 = HEADS * HID
EDGE_DIM = 5
G = 64

_RB = 1000
_EB = 2000
_CH = 128
_NW = 32
E_PAD = 163840
_CPW = E_PAD // _NW // _CH

_EPW = E // _NW
_SFULL = _EPW // _CH
_STAIL = _EPW - _SFULL * _CH

_W1 = 128
_W2 = 128
N_PAD = 10240
_NPT = N_PAD // 16


def _prologue_body(x_ref, gin_ref, bin_ref, Wl_ref, bl_ref, Wr_ref, br_ref,
                   Wres_ref, bres_ref, x0_ref, xl_ref, xr_ref, res_ref):
    x0 = x_ref[:] * gin_ref[:] + bin_ref[:]
    x0_ref[:] = x0
    xl_ref[:] = jnp.dot(x0, Wl_ref[:], preferred_element_type=jnp.float32) + bl_ref[:]
    xr_ref[:] = jnp.dot(x0, Wr_ref[:], preferred_element_type=jnp.float32) + br_ref[:]
    res_ref[:] = jnp.dot(x0, Wres_ref[:], preferred_element_type=jnp.float32) + bres_ref[:]


def _dense_prologue(x, gin_eff, bin_, Wl1, bl1, Wr1, br1, Wres, bres):
    return pl.pallas_call(
        _prologue_body,
        grid=(N // _RB,),
        in_specs=[
            pl.BlockSpec((_RB, IN_DIM), lambda i: (i, 0)),
            pl.BlockSpec((1, IN_DIM), lambda i: (0, 0)),
            pl.BlockSpec((1, IN_DIM), lambda i: (0, 0)),
            pl.BlockSpec((IN_DIM, H1), lambda i: (0, 0)),
            pl.BlockSpec((1, H1), lambda i: (0, 0)),
            pl.BlockSpec((IN_DIM, H1), lambda i: (0, 0)),
            pl.BlockSpec((1, H1), lambda i: (0, 0)),
            pl.BlockSpec((IN_DIM, HID), lambda i: (0, 0)),
            pl.BlockSpec((1, HID), lambda i: (0, 0)),
        ],
        out_specs=[
            pl.BlockSpec((_RB, IN_DIM), lambda i: (i, 0)),
            pl.BlockSpec((_RB, H1), lambda i: (i, 0)),
            pl.BlockSpec((_RB, H1), lambda i: (i, 0)),
            pl.BlockSpec((_RB, HID), lambda i: (i, 0)),
        ],
        out_shape=[
            jax.ShapeDtypeStruct((N, IN_DIM), jnp.float32),
            jax.ShapeDtypeStruct((N, H1), jnp.float32),
            jax.ShapeDtypeStruct((N, H1), jnp.float32),
            jax.ShapeDtypeStruct((N, HID), jnp.float32),
        ],
    )(x, gin_eff[None, :], bin_[None, :], Wl1, bl1[None, :], Wr1, br1[None, :],
      Wres, bres[None, :])


def _make_gather(DA, DB):
    mesh = plsc.VectorSubcoreMesh(core_axis_name="c", subcore_axis_name="s")

    @functools.partial(
        pl.kernel, mesh=mesh,
        out_type=[
            jax.ShapeDtypeStruct((E_PAD, DA), jnp.float32),
            jax.ShapeDtypeStruct((E_PAD, DB), jnp.float32),
        ],
        scratch_types=[
            pltpu.VMEM((_CH,), jnp.int32),
            pltpu.VMEM((_CH,), jnp.int32),
            pltpu.VMEM((_CH, DA), jnp.float32),
            pltpu.VMEM((_CH, DA), jnp.float32),
            pltpu.SemaphoreType.DMA,
            pltpu.SemaphoreType.DMA,
        ],
    )
    def k(ta_hbm, ia_hbm, tb_hbm, ib_hbm, oa_hbm, ob_hbm,
          idx0_v, idx1_v, a0_v, a1_v, sem0, sem1):
        wid = lax.axis_index("s") * 2 + lax.axis_index("c")
        wbase = wid * (_CPW * _CH)

        def run_table(t_hbm, i_hbm, o_hbm, buf0, buf1):
            pltpu.sync_copy(i_hbm.at[pl.ds(wbase, _CH)], idx0_v)
            pltpu.async_copy(t_hbm.at[idx0_v], buf0, sem0)

            def body(kk, _):
                c0 = kk * 2
                base0 = wbase + c0 * _CH
                base1 = base0 + _CH
                pltpu.sync_copy(i_hbm.at[pl.ds(base1, _CH)], idx1_v)
                pltpu.async_copy(t_hbm.at[idx1_v], buf1, sem1)
                pltpu.make_async_copy(t_hbm.at[idx0_v], buf0, sem0).wait()
                pltpu.sync_copy(buf0, o_hbm.at[pl.ds(base0, _CH)])

                @pl.when(kk < _CPW // 2 - 1)
                def _():
                    pltpu.sync_copy(i_hbm.at[pl.ds(base1 + _CH, _CH)], idx0_v)
                    pltpu.async_copy(t_hbm.at[idx0_v], buf0, sem0)

                pltpu.make_async_copy(t_hbm.at[idx1_v], buf1, sem1).wait()
                pltpu.sync_copy(buf1, o_hbm.at[pl.ds(base1, _CH)])
                return ()

            lax.fori_loop(0, _CPW // 2, body, (), unroll=False)

        run_table(ta_hbm, ia_hbm, oa_hbm, a0_v, a1_v)
        run_table(tb_hbm, ib_hbm, ob_hbm, a0_v, a1_v)

    return k


_gather1 = _make_gather(H1, H1)
_gather2 = _make_gather(2 * HID, 2 * HID)


def _scatter1(xjwp0, xjwp1, xjwp2, dst, zeros_hbm):
    mesh = plsc.VectorSubcoreMesh(core_axis_name="c", subcore_axis_name="s")

    @functools.partial(
        pl.kernel, mesh=mesh,
        out_type=jax.ShapeDtypeStruct((2, 3, N_PAD, _W1), jnp.float32),
        scratch_types=[
            pltpu.VMEM((_CH,), jnp.int32),
            pltpu.VMEM((_CH, _W1), jnp.float32),
            pltpu.VMEM((_STAIL,), jnp.int32),
            pltpu.VMEM((_STAIL, _W1), jnp.float32),
            pltpu.VMEM_SHARED((N_PAD, _W1), jnp.float32),
            pltpu.SemaphoreType.DMA,
        ],
    )
    def k(p0_hbm, p1_hbm, p2_hbm, d_hbm, z_hbm, out_hbm, idx_v, buf_v,
          idxt_v, buft_v, acc_sh, sem):
        cid = lax.axis_index("c")
        sid = lax.axis_index("s")
        wid = sid * 2 + cid
        wbase = wid * _EPW

        for ph, src_hbm in ((0, p0_hbm), (1, p1_hbm), (2, p2_hbm)):
            pltpu.sync_copy(z_hbm.at[pl.ds(sid * _NPT, _NPT)],
                            acc_sh.at[pl.ds(sid * _NPT, _NPT)])
            plsc.subcore_barrier()

            def body(c, _):
                base = wbase + c * _CH
                pltpu.sync_copy(d_hbm.at[pl.ds(base, _CH)], idx_v)
                pltpu.sync_copy(src_hbm.at[pl.ds(base, _CH)], buf_v)
                pltpu.sync_copy(buf_v, acc_sh.at[idx_v], add=True)
                return ()

            lax.fori_loop(0, _SFULL, body, (), unroll=False)

            tbase = wbase + _SFULL * _CH
            pltpu.sync_copy(d_hbm.at[pl.ds(tbase, _STAIL)], idxt_v)
            pltpu.sync_copy(src_hbm.at[pl.ds(tbase, _STAIL)], buft_v)
            pltpu.sync_copy(buft_v, acc_sh.at[idxt_v], add=True)

            plsc.subcore_barrier()
            pltpu.sync_copy(acc_sh.at[pl.ds(sid * _NPT, _NPT)],
                            out_hbm.at[cid, ph, pl.ds(sid * _NPT, _NPT)])
            plsc.subcore_barrier()

    return k(xjwp0, xjwp1, xjwp2, dst, zeros_hbm)


def _scatter2(xjwp2, dst, zeros_hbm):
    mesh = plsc.VectorSubcoreMesh(core_axis_name="c", subcore_axis_name="s")

    @functools.partial(
        pl.kernel, mesh=mesh,
        out_type=jax.ShapeDtypeStruct((2, N_PAD, _W2), jnp.float32),
        scratch_types=[
            pltpu.VMEM((_CH,), jnp.int32),
            pltpu.VMEM((_CH, _W2), jnp.float32),
            pltpu.VMEM((_STAIL,), jnp.int32),
            pltpu.VMEM((_STAIL, _W2), jnp.float32),
            pltpu.VMEM_SHARED((N_PAD, _W2), jnp.float32),
            pltpu.SemaphoreType.DMA,
        ],
    )
    def k(p_hbm, d_hbm, z_hbm, out_hbm, idx_v, buf_v, idxt_v, buft_v,
          acc_sh, sem):
        cid = lax.axis_index("c")
        sid = lax.axis_index("s")
        wid = sid * 2 + cid
        wbase = wid * _EPW

        pltpu.sync_copy(z_hbm.at[pl.ds(sid * _NPT, _NPT)],
                        acc_sh.at[pl.ds(sid * _NPT, _NPT)])
        plsc.subcore_barrier()

        def body(c, _):
            base = wbase + c * _CH
            pltpu.sync_copy(d_hbm.at[pl.ds(base, _CH)], idx_v)
            pltpu.sync_copy(p_hbm.at[pl.ds(base, _CH)], buf_v)
            pltpu.sync_copy(buf_v, acc_sh.at[idx_v], add=True)
            return ()

        lax.fori_loop(0, _SFULL, body, (), unroll=False)

        tbase = wbase + _SFULL * _CH
        pltpu.sync_copy(d_hbm.at[pl.ds(tbase, _STAIL)], idxt_v)
        pltpu.sync_copy(p_hbm.at[pl.ds(tbase, _STAIL)], buft_v)
        pltpu.sync_copy(buft_v, acc_sh.at[idxt_v], add=True)

        plsc.subcore_barrier()
        pltpu.sync_copy(acc_sh.at[pl.ds(sid * _NPT, _NPT)],
                        out_hbm.at[cid, pl.ds(sid * _NPT, _NPT)])
        plsc.subcore_barrier()

    return k(xjwp2, dst, zeros_hbm)


def _edge1_body(xj_ref, xi_ref, ea_ref, We_ref, att_ref, p0_ref, p1_ref, p2_ref):
    e = jnp.dot(ea_ref[:, :EDGE_DIM], We_ref[:],
                preferred_element_type=jnp.float32)
    z = xj_ref[:] + xi_ref[:] + e
    z = jnp.where(z > 0, z, 0.2 * z)
    B = z.shape[0]
    alpha = (z.reshape(B, HEADS, HID) * att_ref[:][None]).sum(-1)
    ex = jnp.exp(alpha)
    xjw = (ex[:, :, None] * xj_ref[:].reshape(B, HEADS, HID)).reshape(B, H1)
    p0_ref[:] = xjw[:, :128]
    p1_ref[:] = xjw[:, 128:]
    p2_ref[:] = jnp.concatenate(
        [ex, jnp.zeros((B, _W1 - HEADS), jnp.float32)], axis=1)


def _edge1(xj, xi, ea8, We1, att1):
    return pl.pallas_call(
        _edge1_body,
        grid=(E // _EB,),
        in_specs=[
            pl.BlockSpec((_EB, H1), lambda i: (i, 0)),
            pl.BlockSpec((_EB, H1), lambda i: (i, 0)),
            pl.BlockSpec((_EB, 8), lambda i: (i, 0)),
            pl.BlockSpec((EDGE_DIM, H1), lambda i: (0, 0)),
            pl.BlockSpec((HEADS, HID), lambda i: (0, 0)),
        ],
        out_specs=[
            pl.BlockSpec((_EB, _W1), lambda i: (i, 0)),
            pl.BlockSpec((_EB, _W1), lambda i: (i, 0)),
            pl.BlockSpec((_EB, _W1), lambda i: (i, 0)),
        ],
        out_shape=[
            jax.ShapeDtypeStruct((E, _W1), jnp.float32),
            jax.ShapeDtypeStruct((E, _W1), jnp.float32),
            jax.ShapeDtypeStruct((E, _W1), jnp.float32),
        ],
    )(xj, xi, ea8, We1, att1)


def _mid_body(p00_ref, p10_ref, p01_ref, p11_ref, p02_ref, p12_ref,
              xl1_ref, xr1_ref, eal_ref,
              We1_ref, att1_ref, bias_ref, g_ref, b_ref, Wl_ref, bl_ref,
              Wr_ref, br_ref, xb2_ref):
    B = p00_ref.shape[2]
    c0 = p00_ref[0, 0] + p10_ref[0, 0]
    c1 = p01_ref[0, 0] + p11_ref[0, 0]
    raw = jnp.concatenate([c0, c1], axis=1)
    s_edges = (p02_ref[0, 0] + p12_ref[0, 0])[:, :HEADS]
    el = jnp.dot(eal_ref[:], We1_ref[:], preferred_element_type=jnp.float32)
    zs = xl1_ref[:] + xr1_ref[:] + el
    zs = jnp.where(zs > 0, zs, 0.2 * zs)
    exs = jnp.exp((zs.reshape(B, HEADS, HID) * att1_ref[:][None]).sum(-1))
    tot = raw.reshape(B, HEADS, HID) + exs[:, :, None] * xl1_ref[:].reshape(B, HEADS, HID)
    s = s_edges + exs + 1e-16
    h = (tot / s[:, :, None]).reshape(B, H1) + bias_ref[:]
    mu = h.mean(-1, keepdims=True)
    var = ((h - mu) ** 2).mean(-1, keepdims=True)
    hn = (h - mu) / jnp.sqrt(var + 1e-5) * g_ref[:] + b_ref[:]
    he = jnp.where(hn > 0, hn, jnp.exp(hn) - 1.0)
    xl2 = jnp.dot(he, Wl_ref[:], preferred_element_type=jnp.float32) + bl_ref[:]
    xr2 = jnp.dot(he, Wr_ref[:], preferred_element_type=jnp.float32) + br_ref[:]
    xb2_ref[:] = jnp.concatenate([xl2, xr2], axis=1)


def _mid(P1, xl1, xr1, eal, We1, att1, bias1, g1, b1, Wl2, bl2, Wr2, br2):
    return pl.pallas_call(
        _mid_body,
        grid=(N // _RB,),
        in_specs=[
            pl.BlockSpec((1, 1, _RB, _W1), lambda i: (0, 0, i, 0)),
            pl.BlockSpec((1, 1, _RB, _W1), lambda i: (1, 0, i, 0)),
            pl.BlockSpec((1, 1, _RB, _W1), lambda i: (0, 1, i, 0)),
            pl.BlockSpec((1, 1, _RB, _W1), lambda i: (1, 1, i, 0)),
            pl.BlockSpec((1, 1, _RB, _W1), lambda i: (0, 2, i, 0)),
            pl.BlockSpec((1, 1, _RB, _W1), lambda i: (1, 2, i, 0)),
            pl.BlockSpec((_RB, H1), lambda i: (i, 0)),
            pl.BlockSpec((_RB, H1), lambda i: (i, 0)),
            pl.BlockSpec((_RB, EDGE_DIM), lambda i: (i, 0)),
            pl.BlockSpec((EDGE_DIM, H1), lambda i: (0, 0)),
            pl.BlockSpec((HEADS, HID), lambda i: (0, 0)),
            pl.BlockSpec((1, H1), lambda i: (0, 0)),
            pl.BlockSpec((1, H1), lambda i: (0, 0)),
            pl.BlockSpec((1, H1), lambda i: (0, 0)),
            pl.BlockSpec((H1, HID), lambda i: (0, 0)),
            pl.BlockSpec((1, HID), lambda i: (0, 0)),
            pl.BlockSpec((H1, HID), lambda i: (0, 0)),
            pl.BlockSpec((1, HID), lambda i: (0, 0)),
        ],
        out_specs=pl.BlockSpec((_RB, 2 * HID), lambda i: (i, 0)),
        out_shape=jax.ShapeDtypeStruct((N, 2 * HID), jnp.float32),
    )(P1, P1, P1, P1, P1, P1, xl1, xr1, eal, We1, att1, bias1[None, :],
      g1[None, :], b1[None, :], Wl2, bl2[None, :], Wr2, br2[None, :])


def _mid_wrap(P1, *args):
    return _mid(P1, *args)


def _edge2_body(xjf_ref, xif_ref, ea_ref, We_ref, att_ref, xjw_ref):
    e = jnp.dot(ea_ref[:, :EDGE_DIM], We_ref[:],
                preferred_element_type=jnp.float32)
    xj = xjf_ref[:, :HID]
    xi = xif_ref[:, HID:]
    z = xj + xi + e
    z = jnp.where(z > 0, z, 0.2 * z)
    alpha = (z * att_ref[:]).sum(-1, keepdims=True)
    ex = jnp.exp(alpha)
    B = z.shape[0]
    xjw_ref[:] = jnp.concatenate(
        [ex * xj, ex, jnp.zeros((B, _W2 - HID - 1), jnp.float32)], axis=1)


def _edge2(xjf, xif, ea8, We2, att2):
    return pl.pallas_call(
        _edge2_body,
        grid=(E // _EB,),
        in_specs=[
            pl.BlockSpec((_EB, 2 * HID), lambda i: (i, 0)),
            pl.BlockSpec((_EB, 2 * HID), lambda i: (i, 0)),
            pl.BlockSpec((_EB, 8), lambda i: (i, 0)),
            pl.BlockSpec((EDGE_DIM, HID), lambda i: (0, 0)),
            pl.BlockSpec((1, HID), lambda i: (0, 0)),
        ],
        out_specs=pl.BlockSpec((_EB, _W2), lambda i: (i, 0)),
        out_shape=jax.ShapeDtypeStruct((E, _W2), jnp.float32),
    )(xjf, xif, ea8, We2, att2)


def _epi_body(q0_ref, q1_ref, xb2_ref, eal_ref, We2_ref, att2_ref, bias_ref,
              g_ref, b_ref, res_ref, out_ref):
    raw = q0_ref[0] + q1_ref[0]
    xl2 = xb2_ref[:, :HID]
    xr2 = xb2_ref[:, HID:]
    el = jnp.dot(eal_ref[:], We2_ref[:], preferred_element_type=jnp.float32)
    zs = xl2 + xr2 + el
    zs = jnp.where(zs > 0, zs, 0.2 * zs)
    exs = jnp.exp((zs * att2_ref[:]).sum(-1, keepdims=True))
    tot = raw[:, :HID] + exs * xl2
    s = raw[:, HID:HID + 1] + exs + 1e-16
    h = tot / s + bias_ref[:]
    mu = h.mean(-1, keepdims=True)
    var = ((h - mu) ** 2).mean(-1, keepdims=True)
    hn = (h - mu) / jnp.sqrt(var + 1e-5) * g_ref[:] + b_ref[:]
    he = jnp.where(hn > 0, hn, jnp.exp(hn) - 1.0)
    out_ref[:] = he + res_ref[:]


def _epilogue(P2, xb2, eal, We2, att2, bias2, g2, b2n, res):
    return pl.pallas_call(
        _epi_body,
        grid=(N // _RB,),
        in_specs=[
            pl.BlockSpec((1, _RB, _W2), lambda i: (0, i, 0)),
            pl.BlockSpec((1, _RB, _W2), lambda i: (1, i, 0)),
            pl.BlockSpec((_RB, 2 * HID), lambda i: (i, 0)),
            pl.BlockSpec((_RB, EDGE_DIM), lambda i: (i, 0)),
            pl.BlockSpec((EDGE_DIM, HID), lambda i: (0, 0)),
            pl.BlockSpec((1, HID), lambda i: (0, 0)),
            pl.BlockSpec((1, HID), lambda i: (0, 0)),
            pl.BlockSpec((1, HID), lambda i: (0, 0)),
            pl.BlockSpec((1, HID), lambda i: (0, 0)),
            pl.BlockSpec((_RB, HID), lambda i: (i, 0)),
        ],
        out_specs=pl.BlockSpec((_RB, HID), lambda i: (i, 0)),
        out_shape=jax.ShapeDtypeStruct((N, HID), jnp.float32),
    )(P2, P2, xb2, eal, We2, att2[0:1], bias2[None, :], g2[None, :],
      b2n[None, :], res)


def kernel(x, edge_index, edge_attr, batch, gin, bin_, Wl1, bl1, Wr1, br1, We1,
           att1, bias1, g1, b1, Wl2, bl2, Wr2, br2, We2, att2, bias2, g2, b2n,
           Wres, bres):
    src = edge_index[0]
    dst = edge_index[1]
    pad = jnp.zeros((E_PAD - E,), jnp.int32)
    srcp = jnp.concatenate([src, pad])
    dstp = jnp.concatenate([dst, pad])
    zeros_nw1 = jnp.zeros((N_PAD, _W1), jnp.float32)

    deg = jax.ops.segment_sum(jnp.ones((E,), jnp.float32), dst, num_segments=N)
    eal = jax.ops.segment_sum(edge_attr, dst, num_segments=N)
    eal = eal / jnp.maximum(deg, 1.0)[:, None]

    ea8 = jnp.concatenate([edge_attr, jnp.zeros((E, 3), jnp.float32)], axis=1)

    gin_eff = gin / jnp.sqrt(jnp.float32(1.0 + 1e-5))
    x0, xl1, xr1, res = _dense_prologue(x, gin_eff, bin_, Wl1, bl1, Wr1, br1,
                                        Wres, bres)

    xj1, xi1 = _gather1(xl1, srcp, xr1, dstp)
    xjwp0, xjwp1, xjwp2 = _edge1(xj1, xi1, ea8, We1, att1)
    P1 = _scatter1(xjwp0, xjwp1, xjwp2, dst, zeros_nw1)
    xb2 = _mid_wrap(P1, xl1, xr1, eal, We1, att1, bias1, g1, b1, Wl2, bl2,
                    Wr2, br2)

    xjf2, xif2 = _gather2(xb2, srcp, xb2, dstp)
    xjw2 = _edge2(xjf2, xif2, ea8, We2, att2)
    P2 = _scatter2(xjw2, dst, zeros_nw1)
    node_emb = _epilogue(P2, xb2, eal, We2, att2, bias2, g2, b2n, res)

    cnt = jax.ops.segment_sum(jnp.ones((N,), jnp.float32), batch, num_segments=G)
    mean = jax.ops.segment_sum(node_emb, batch, num_segments=G) / jnp.maximum(cnt, 1.0)[:, None]
    mx = jax.ops.segment_max(node_emb, batch, num_segments=G)
    mx = jnp.where(jnp.isfinite(mx), mx, 0.0)
    graph_emb = jnp.concatenate([mean, mx], axis=-1)
    return node_emb, graph_emb
